# baseline (speedup 1.0000x reference)
import functools

import jax
import jax.numpy as jnp
from jax import lax
from jax.experimental import pallas as pl
from jax.experimental.pallas import tpu as pltpu
from jax.experimental.pallas import tpu_sc as plsc

_POOL = 10
_LEN = 20
_D = 768
_B = 512
_ROW = 2 * _LEN * _D

_NC = 2
_NS = 16
_NW = _NC * _NS
_BPW = _B // _NW
_NCHUNK = 10
_DC = _ROW // _NCHUNK
_NBUF = 2


def _table_body(prompt_ref, wk1_ref, bk1_ref, wk2_ref, bk2_ref,
                wv1_ref, bv1_ref, wv2_ref, bv2_ref, out_ref):
    p0 = prompt_ref[:, 0, 0].reshape(_POOL * _LEN, _D)
    h0 = jnp.maximum(
        jnp.dot(p0, wk1_ref[...], preferred_element_type=jnp.float32)
        + bk1_ref[...], 0.0)
    t0 = p0 + jnp.dot(h0, wk2_ref[...], preferred_element_type=jnp.float32) \
        + bk2_ref[...]
    out_ref[:, 0:_LEN, :] = t0.reshape(_POOL, _LEN, _D)
    p1 = prompt_ref[:, 1, 0].reshape(_POOL * _LEN, _D)
    h1 = jnp.maximum(
        jnp.dot(p1, wv1_ref[...], preferred_element_type=jnp.float32)
        + bv1_ref[...], 0.0)
    t1 = p1 + jnp.dot(h1, wv2_ref[...], preferred_element_type=jnp.float32) \
        + bv2_ref[...]
    out_ref[:, _LEN:2 * _LEN, :] = t1.reshape(_POOL, _LEN, _D)


def _build_table(prompt, Wk1, bk1, Wk2, bk2, Wv1, bv1, Wv2, bv2):
    return pl.pallas_call(
        _table_body,
        out_shape=jax.ShapeDtypeStruct((_POOL, 2 * _LEN, _D), jnp.float32),
    )(prompt, Wk1, bk1, Wk2, bk2, Wv1, bv1, Wv2, bv2)


_NSLAB = 2
_CPS = _NCHUNK // _NSLAB


def _gather_body(c0, table_ref, idx_ref, out_ref, idx_v, sidx, bufs,
                 gsems, wsems):
    wid = lax.axis_index("s") * _NC + lax.axis_index("c")
    base = wid * _BPW
    pltpu.sync_copy(idx_ref.at[pl.ds(base, _BPW)], idx_v)
    idx = idx_v[...]
    gd = [None] * _NBUF
    wd = [None] * _NBUF

    def dst(c):
        return out_ref.at[pl.ds(base, _BPW), pl.ds((c - c0) * _DC, _DC)]

    for c in range(c0, c0 + _CPS):
        s = c % _NBUF
        if wd[s] is not None:
            wd[s].wait()
        sidx[s][...] = idx * _NCHUNK + c
        gd[s] = pltpu.async_copy(table_ref.at[sidx[s]], bufs[s], gsems[s])
        if c > c0:
            p = (c - 1) % _NBUF
            gd[p].wait()
            wd[p] = pltpu.async_copy(bufs[p], dst(c - 1), wsems[p])
    sl = (c0 + _CPS - 1) % _NBUF
    gd[sl].wait()
    wd[sl] = pltpu.async_copy(bufs[sl], dst(c0 + _CPS - 1), wsems[sl])
    for w in wd:
        if w is not None:
            w.wait()


def _gather(table2, idx, c0):
    mesh = plsc.VectorSubcoreMesh(
        core_axis_name="c", subcore_axis_name="s",
        num_cores=_NC, num_subcores=_NS)
    run = functools.partial(
        pl.kernel,
        out_type=jax.ShapeDtypeStruct((_B, _CPS * _DC), jnp.float32),
        mesh=mesh,
        scratch_types=[
            pltpu.VMEM((_BPW,), jnp.int32),
            [pltpu.VMEM((_BPW,), jnp.int32)] * _NBUF,
            [pltpu.VMEM((_BPW, _DC), jnp.float32)] * _NBUF,
            [pltpu.SemaphoreType.DMA] * _NBUF,
            [pltpu.SemaphoreType.DMA] * _NBUF,
        ],
    )(functools.partial(_gather_body, c0))
    return run(table2, idx)


def kernel(x_embed, prompt, Wk1, bk1, Wk2, bk2, Wv1, bv1, Wv2, bv2,
           prompt_idx):
    del x_embed
    table = _build_table(prompt, Wk1, bk1, Wk2, bk2, Wv1, bv1, Wv2, bv2)
    table2 = table.reshape(_POOL * _NCHUNK, _DC)
    idx = prompt_idx.astype(jnp.int32)
    slabs = [
        _gather(table2, idx, k * _CPS).reshape(_B, 1, 1, _LEN, 12, 64)
        for k in range(_NSLAB)
    ]
    bp = jnp.concatenate(slabs, axis=2)
    return (prompt_idx, bp)


# speedup vs baseline: 1.3021x; 1.3021x over previous
import functools

import jax
import jax.numpy as jnp
from jax import lax
from jax.experimental import pallas as pl
from jax.experimental.pallas import tpu as pltpu
from jax.experimental.pallas import tpu_sc as plsc

_POOL = 10
_LEN = 20
_D = 768
_B = 512
_ROW = 2 * _LEN * _D

_NC = 2
_NS = 16
_NW = _NC * _NS
_BPW = _B // _NW
_NCHUNK = 10
_DC = _ROW // _NCHUNK
_NBUF = 2


def _table_body(prompt_ref, wk1_ref, bk1_ref, wk2_ref, bk2_ref,
                wv1_ref, bv1_ref, wv2_ref, bv2_ref, out_ref):
    p0 = prompt_ref[:, 0, 0].reshape(_POOL * _LEN, _D)
    h0 = jnp.maximum(
        jnp.dot(p0, wk1_ref[...], preferred_element_type=jnp.float32)
        + bk1_ref[...], 0.0)
    t0 = p0 + jnp.dot(h0, wk2_ref[...], preferred_element_type=jnp.float32) \
        + bk2_ref[...]
    out_ref[:, 0:_LEN, :] = t0.reshape(_POOL, _LEN, _D)
    p1 = prompt_ref[:, 1, 0].reshape(_POOL * _LEN, _D)
    h1 = jnp.maximum(
        jnp.dot(p1, wv1_ref[...], preferred_element_type=jnp.float32)
        + bv1_ref[...], 0.0)
    t1 = p1 + jnp.dot(h1, wv2_ref[...], preferred_element_type=jnp.float32) \
        + bv2_ref[...]
    out_ref[:, _LEN:2 * _LEN, :] = t1.reshape(_POOL, _LEN, _D)


def _build_table(prompt, Wk1, bk1, Wk2, bk2, Wv1, bv1, Wv2, bv2):
    return pl.pallas_call(
        _table_body,
        out_shape=jax.ShapeDtypeStruct((_POOL, 2 * _LEN, _D), jnp.float32),
    )(prompt, Wk1, bk1, Wk2, bk2, Wv1, bv1, Wv2, bv2)


_NSLAB = 1
_CPS = _NCHUNK // _NSLAB


def _gather_body(c0, table_ref, idx_ref, out_ref, idx_v, sidx, bufs,
                 gsems, wsems):
    wid = lax.axis_index("s") * _NC + lax.axis_index("c")
    base = wid * _BPW
    pltpu.sync_copy(idx_ref.at[pl.ds(base, _BPW)], idx_v)
    idx = idx_v[...]
    gd = [None] * _NBUF
    wd = [None] * _NBUF

    def dst(c):
        return out_ref.at[pl.ds(base, _BPW), pl.ds((c - c0) * _DC, _DC)]

    for c in range(c0, c0 + _CPS):
        s = c % _NBUF
        if wd[s] is not None:
            wd[s].wait()
        sidx[s][...] = idx * _NCHUNK + c
        gd[s] = pltpu.async_copy(table_ref.at[sidx[s]], bufs[s], gsems[s])
        if c > c0:
            p = (c - 1) % _NBUF
            gd[p].wait()
            wd[p] = pltpu.async_copy(bufs[p], dst(c - 1), wsems[p])
    sl = (c0 + _CPS - 1) % _NBUF
    gd[sl].wait()
    wd[sl] = pltpu.async_copy(bufs[sl], dst(c0 + _CPS - 1), wsems[sl])
    for w in wd:
        if w is not None:
            w.wait()


def _gather(table2, idx, c0):
    mesh = plsc.VectorSubcoreMesh(
        core_axis_name="c", subcore_axis_name="s",
        num_cores=_NC, num_subcores=_NS)
    run = functools.partial(
        pl.kernel,
        out_type=jax.ShapeDtypeStruct((_B, _CPS * _DC), jnp.float32),
        mesh=mesh,
        scratch_types=[
            pltpu.VMEM((_BPW,), jnp.int32),
            [pltpu.VMEM((_BPW,), jnp.int32)] * _NBUF,
            [pltpu.VMEM((_BPW, _DC), jnp.float32)] * _NBUF,
            [pltpu.SemaphoreType.DMA] * _NBUF,
            [pltpu.SemaphoreType.DMA] * _NBUF,
        ],
    )(functools.partial(_gather_body, c0))
    return run(table2, idx)


def kernel(x_embed, prompt, Wk1, bk1, Wk2, bk2, Wv1, bv1, Wv2, bv2,
           prompt_idx):
    del x_embed
    table = _build_table(prompt, Wk1, bk1, Wk2, bk2, Wv1, bv1, Wv2, bv2)
    table2 = table.reshape(_POOL * _NCHUNK, _DC)
    idx = prompt_idx.astype(jnp.int32)
    out = _gather(table2, idx, 0)
    bp = out.reshape(_B, 1, 2, _LEN, 12, 64)
    return (prompt_idx, bp)


# speedup vs baseline: 1.3098x; 1.0059x over previous
import functools

import jax
import jax.numpy as jnp
from jax import lax
from jax.experimental import pallas as pl
from jax.experimental.pallas import tpu as pltpu
from jax.experimental.pallas import tpu_sc as plsc

_POOL = 10
_LEN = 20
_D = 768
_B = 512
_ROW = 2 * _LEN * _D

_NC = 2
_NS = 16
_NW = _NC * _NS
_BPW = _B // _NW
_NCHUNK = 12
_DC = _ROW // _NCHUNK
_NBUF = 3


def _table_body(prompt_ref, wk1_ref, bk1_ref, wk2_ref, bk2_ref,
                wv1_ref, bv1_ref, wv2_ref, bv2_ref, out_ref):
    p0 = prompt_ref[:, 0, 0].reshape(_POOL * _LEN, _D)
    h0 = jnp.maximum(
        jnp.dot(p0, wk1_ref[...], preferred_element_type=jnp.float32)
        + bk1_ref[...], 0.0)
    t0 = p0 + jnp.dot(h0, wk2_ref[...], preferred_element_type=jnp.float32) \
        + bk2_ref[...]
    out_ref[:, 0:_LEN, :] = t0.reshape(_POOL, _LEN, _D)
    p1 = prompt_ref[:, 1, 0].reshape(_POOL * _LEN, _D)
    h1 = jnp.maximum(
        jnp.dot(p1, wv1_ref[...], preferred_element_type=jnp.float32)
        + bv1_ref[...], 0.0)
    t1 = p1 + jnp.dot(h1, wv2_ref[...], preferred_element_type=jnp.float32) \
        + bv2_ref[...]
    out_ref[:, _LEN:2 * _LEN, :] = t1.reshape(_POOL, _LEN, _D)


def _build_table(prompt, Wk1, bk1, Wk2, bk2, Wv1, bv1, Wv2, bv2):
    return pl.pallas_call(
        _table_body,
        out_shape=jax.ShapeDtypeStruct((_POOL, 2 * _LEN, _D), jnp.float32),
    )(prompt, Wk1, bk1, Wk2, bk2, Wv1, bv1, Wv2, bv2)


_NSLAB = 1
_CPS = _NCHUNK // _NSLAB


def _gather_body(c0, table_ref, idx_ref, out_ref, idx_v, sidx, bufs,
                 gsems, wsems):
    wid = lax.axis_index("s") * _NC + lax.axis_index("c")
    base = wid * _BPW
    pltpu.sync_copy(idx_ref.at[pl.ds(base, _BPW)], idx_v)
    idx = idx_v[...]
    gd = [None] * _NBUF
    wd = [None] * _NBUF

    def dst(c):
        return out_ref.at[pl.ds(base, _BPW), pl.ds((c - c0) * _DC, _DC)]

    for c in range(c0, c0 + _CPS):
        s = c % _NBUF
        if wd[s] is not None:
            wd[s].wait()
        sidx[s][...] = idx * _NCHUNK + c
        gd[s] = pltpu.async_copy(table_ref.at[sidx[s]], bufs[s], gsems[s])
        if c > c0:
            p = (c - 1) % _NBUF
            gd[p].wait()
            wd[p] = pltpu.async_copy(bufs[p], dst(c - 1), wsems[p])
    sl = (c0 + _CPS - 1) % _NBUF
    gd[sl].wait()
    wd[sl] = pltpu.async_copy(bufs[sl], dst(c0 + _CPS - 1), wsems[sl])
    for w in wd:
        if w is not None:
            w.wait()


def _gather(table2, idx, c0):
    mesh = plsc.VectorSubcoreMesh(
        core_axis_name="c", subcore_axis_name="s",
        num_cores=_NC, num_subcores=_NS)
    run = functools.partial(
        pl.kernel,
        out_type=jax.ShapeDtypeStruct((_B, _CPS * _DC), jnp.float32),
        mesh=mesh,
        scratch_types=[
            pltpu.VMEM((_BPW,), jnp.int32),
            [pltpu.VMEM((_BPW,), jnp.int32)] * _NBUF,
            [pltpu.VMEM((_BPW, _DC), jnp.float32)] * _NBUF,
            [pltpu.SemaphoreType.DMA] * _NBUF,
            [pltpu.SemaphoreType.DMA] * _NBUF,
        ],
    )(functools.partial(_gather_body, c0))
    return run(table2, idx)


def kernel(x_embed, prompt, Wk1, bk1, Wk2, bk2, Wv1, bv1, Wv2, bv2,
           prompt_idx):
    del x_embed
    table = _build_table(prompt, Wk1, bk1, Wk2, bk2, Wv1, bv1, Wv2, bv2)
    table2 = table.reshape(_POOL * _NCHUNK, _DC)
    idx = prompt_idx.astype(jnp.int32)
    out = _gather(table2, idx, 0)
    bp = out.reshape(_B, 1, 2, _LEN, 12, 64)
    return (prompt_idx, bp)


# speedup vs baseline: 1.3215x; 1.0090x over previous
import functools

import jax
import jax.numpy as jnp
from jax import lax
from jax.experimental import pallas as pl
from jax.experimental.pallas import tpu as pltpu
from jax.experimental.pallas import tpu_sc as plsc

_POOL = 10
_LEN = 20
_D = 768
_B = 512
_ROW = 2 * _LEN * _D

_NC = 2
_NS = 16
_NW = _NC * _NS
_BPW = _B // _NW
_NCHUNK = 12
_DC = _ROW // _NCHUNK
_NBUF = 3


def _table_body(prompt_ref, wk1_ref, bk1_ref, wk2_ref, bk2_ref,
                wv1_ref, bv1_ref, wv2_ref, bv2_ref, out_ref):
    p0 = prompt_ref[:, 0, 0].reshape(_POOL * _LEN, _D)
    h0 = jnp.maximum(
        jnp.dot(p0, wk1_ref[...], preferred_element_type=jnp.float32)
        + bk1_ref[...], 0.0)
    t0 = p0 + jnp.dot(h0, wk2_ref[...], preferred_element_type=jnp.float32) \
        + bk2_ref[...]
    out_ref[:, 0:_LEN, :] = t0.reshape(_POOL, _LEN, _D)
    p1 = prompt_ref[:, 1, 0].reshape(_POOL * _LEN, _D)
    h1 = jnp.maximum(
        jnp.dot(p1, wv1_ref[...], preferred_element_type=jnp.float32)
        + bv1_ref[...], 0.0)
    t1 = p1 + jnp.dot(h1, wv2_ref[...], preferred_element_type=jnp.float32) \
        + bv2_ref[...]
    out_ref[:, _LEN:2 * _LEN, :] = t1.reshape(_POOL, _LEN, _D)


def _build_table(prompt, Wk1, bk1, Wk2, bk2, Wv1, bv1, Wv2, bv2):
    return pl.pallas_call(
        _table_body,
        out_shape=jax.ShapeDtypeStruct((_POOL, 2 * _LEN, _D), jnp.float32),
    )(prompt, Wk1, bk1, Wk2, bk2, Wv1, bv1, Wv2, bv2)


def _gather_body(table_ref, idx_ref, out_ref, idx_v, sidx, bufs,
                 gsems, wsems):
    wid = lax.axis_index("s") * _NC + lax.axis_index("c")
    base = wid * _BPW
    pltpu.sync_copy(idx_ref.at[pl.ds(base, _BPW)], idx_v)
    idx = idx_v[...]
    for c in range(_NCHUNK):
        sidx[c][...] = idx * _NCHUNK + c
    gd = [None] * _NBUF
    wd = [None] * _NBUF

    def dst(c):
        return out_ref.at[pl.ds(base, _BPW), pl.ds(c * _DC, _DC)]

    for c in range(_NCHUNK):
        s = c % _NBUF
        if wd[s] is not None:
            wd[s].wait()
        gd[s] = pltpu.async_copy(table_ref.at[sidx[c]], bufs[s], gsems[s])
        if c >= 1:
            p = (c - 1) % _NBUF
            gd[p].wait()
            wd[p] = pltpu.async_copy(bufs[p], dst(c - 1), wsems[p])
    sl = (_NCHUNK - 1) % _NBUF
    gd[sl].wait()
    wd[sl] = pltpu.async_copy(bufs[sl], dst(_NCHUNK - 1), wsems[sl])
    for w in wd:
        if w is not None:
            w.wait()


def _gather(table2, idx):
    mesh = plsc.VectorSubcoreMesh(
        core_axis_name="c", subcore_axis_name="s",
        num_cores=_NC, num_subcores=_NS)
    run = functools.partial(
        pl.kernel,
        out_type=jax.ShapeDtypeStruct((_B, _ROW), jnp.float32),
        mesh=mesh,
        scratch_types=[
            pltpu.VMEM((_BPW,), jnp.int32),
            [pltpu.VMEM((_BPW,), jnp.int32)] * _NCHUNK,
            [pltpu.VMEM((_BPW, _DC), jnp.float32)] * _NBUF,
            [pltpu.SemaphoreType.DMA] * _NBUF,
            [pltpu.SemaphoreType.DMA] * _NBUF,
        ],
    )(_gather_body)
    return run(table2, idx)


def kernel(x_embed, prompt, Wk1, bk1, Wk2, bk2, Wv1, bv1, Wv2, bv2,
           prompt_idx):
    del x_embed
    table = _build_table(prompt, Wk1, bk1, Wk2, bk2, Wv1, bv1, Wv2, bv2)
    table2 = table.reshape(_POOL * _NCHUNK, _DC)
    idx = prompt_idx.astype(jnp.int32)
    out = _gather(table2, idx)
    bp = out.reshape(_B, 1, 2, _LEN, 12, 64)
    return (prompt_idx, bp)
